# Initial kernel scaffold; baseline (speedup 1.0000x reference)
import jax
import jax.numpy as jnp
from jax.experimental import pallas as pl


def kernel(img_pad_value, conv_w_t, conv_b_col, vobs_w, vobs_b, tobs_w, tobs_b, lstm_w_ih_v, lstm_w_ih_t, lstm_w_hh, lstm_b, ac_w, ac_b, image, glove, h0, c0):
    raise NotImplementedError("write your pallas kernel here")


# speedup vs baseline: 3.4027x; 3.4027x over previous
import jax
import jax.numpy as jnp
from jax.experimental import pallas as pl
from jax.experimental.pallas import tpu as pltpu

_VMEM = pl.BlockSpec(memory_space=pltpu.MemorySpace.VMEM)


def _make_fused(B, Cout, P, Hd):
    def _fused(pat_ref, wconv_ref, bias_ref, vw_ref, vb_ref,
               glove_ref, tw_ref, tb_ref, h0_ref, c0_ref,
               wihv_ref, wiht_ref, whh_ref, bl_ref, acw_ref, acb_ref,
               h_out_ref, c_out_ref, pv_ref):
        conv = jnp.dot(wconv_ref[...], pat_ref[...],
                       preferred_element_type=jnp.float32)
        conv = conv.reshape(Cout, B, P) + bias_ref[...][:, None, :]
        conv = jnp.maximum(conv, 0.0).astype(jnp.bfloat16)
        acc = jnp.zeros((B, Hd), jnp.float32)
        for c in range(Cout):
            acc = acc + jnp.dot(conv[c], vw_ref[c],
                                preferred_element_type=jnp.float32)
        ve = jnp.maximum(acc + vb_ref[...], 0.0)
        te = jnp.maximum(jnp.dot(glove_ref[...], tw_ref[...],
                                 preferred_element_type=jnp.float32)
                         + tb_ref[...], 0.0)
        gates = (jnp.dot(ve.astype(jnp.bfloat16), wihv_ref[...],
                         preferred_element_type=jnp.float32)
                 + jnp.dot(te.astype(jnp.bfloat16), wiht_ref[...],
                           preferred_element_type=jnp.float32)
                 + jnp.dot(h0_ref[...], whh_ref[...],
                           preferred_element_type=jnp.float32)
                 + bl_ref[...])
        i = jax.nn.sigmoid(gates[:, 0 * Hd:1 * Hd])
        f = jax.nn.sigmoid(gates[:, 1 * Hd:2 * Hd])
        g = jnp.tanh(gates[:, 2 * Hd:3 * Hd])
        o = jax.nn.sigmoid(gates[:, 3 * Hd:4 * Hd])
        c_new = f * c0_ref[...] + i * g
        h_new = o * jnp.tanh(c_new)
        c_out_ref[...] = c_new
        h_out_ref[...] = h_new
        pv_ref[...] = (jnp.dot(h_new.astype(jnp.bfloat16), acw_ref[...],
                               preferred_element_type=jnp.float32)
                       + acb_ref[...])
    return _fused


def kernel(img_pad_value, conv_w_t, conv_b_col, vobs_w, vobs_b, tobs_w, tobs_b,
           lstm_w_ih_v, lstm_w_ih_t, lstm_w_hh, lstm_b, ac_w, ac_b,
           image, glove, h0, c0):
    B, H, W, C = image.shape
    Ho, Wo = H // 2, W // 2
    P = Ho * Wo
    Cout, K = conv_w_t.shape
    Hd = h0.shape[1]
    A1 = ac_w.shape[1]
    bf = jnp.bfloat16

    xp = jnp.pad(image.astype(bf), ((0, 0), (1, 0), (1, 0), (0, 0)))
    slabs = [xp[:, di:di + H:2, dj:dj + W:2, :]
             for di in range(3) for dj in range(3)]
    pat = jnp.stack(slabs, 0)
    pat = pat.transpose(0, 4, 1, 2, 3).reshape(K, B * P)

    ii = jnp.arange(Ho)
    jj = jnp.arange(Wo)
    oob_rows = [((2 * ii + di - 1 < 0) | (2 * ii + di - 1 >= H))
                for di in range(3)]
    oob_cols = [((2 * jj + dj - 1 < 0) | (2 * jj + dj - 1 >= W))
                for dj in range(3)]
    oob = jnp.stack([oob_rows[di][:, None] | oob_cols[dj][None, :]
                     for di in range(3) for dj in range(3)])
    oob27 = jnp.repeat(oob.reshape(9, P).astype(jnp.float32), C, axis=0)
    padv = jnp.tile(img_pad_value.reshape(-1), 9)
    bias = conv_b_col + (conv_w_t * padv[None, :]) @ oob27

    vw3 = vobs_w.reshape(Cout, P, Hd).astype(bf)

    h_new, c_new, pv = pl.pallas_call(
        _make_fused(B, Cout, P, Hd),
        out_shape=(jax.ShapeDtypeStruct((B, Hd), jnp.float32),
                   jax.ShapeDtypeStruct((B, Hd), jnp.float32),
                   jax.ShapeDtypeStruct((B, A1), jnp.float32)),
        in_specs=[_VMEM] * 16,
        out_specs=(_VMEM, _VMEM, _VMEM),
    )(pat, conv_w_t.astype(bf), bias, vw3, vobs_b,
      glove.astype(bf), tobs_w.astype(bf), tobs_b, h0.astype(bf), c0,
      lstm_w_ih_v.astype(bf), lstm_w_ih_t.astype(bf), lstm_w_hh.astype(bf),
      lstm_b, ac_w.astype(bf), ac_b)

    A = A1 - 1
    return {'policy': pv[:, :A], 'value': pv[:, A:], 'hidden': (h_new, c_new)}


# speedup vs baseline: 3.4515x; 1.0143x over previous
import jax
import jax.numpy as jnp
from jax.experimental import pallas as pl
from jax.experimental.pallas import tpu as pltpu

_VMEM = pl.BlockSpec(memory_space=pltpu.MemorySpace.VMEM)


def _make_fused(B, Cout, P, Hd):
    def _fused(pat_ref, wconv_ref, bias_ref, vw_ref, vb_ref,
               glove_ref, tw_ref, tb_ref, h0_ref, c0_ref,
               wihv_ref, wiht_ref, whh_ref, bl_ref, acw_ref, acb_ref,
               h_out_ref, c_out_ref, pv_ref):
        conv = jnp.dot(wconv_ref[...], pat_ref[...],
                       preferred_element_type=jnp.float32)
        conv = conv.reshape(Cout, B, P) + bias_ref[...][:, None, :]
        conv = jnp.maximum(conv, 0.0)
        acc = jnp.zeros((B, Hd), jnp.float32)
        for c in range(Cout):
            acc = acc + jnp.dot(conv[c], vw_ref[c],
                                preferred_element_type=jnp.float32)
        ve = jnp.maximum(acc + vb_ref[...], 0.0)
        te = jnp.maximum(jnp.dot(glove_ref[...], tw_ref[...],
                                 preferred_element_type=jnp.float32)
                         + tb_ref[...], 0.0)
        gates = (jnp.dot(ve, wihv_ref[...],
                         preferred_element_type=jnp.float32)
                 + jnp.dot(te, wiht_ref[...],
                           preferred_element_type=jnp.float32)
                 + jnp.dot(h0_ref[...], whh_ref[...],
                           preferred_element_type=jnp.float32)
                 + bl_ref[...])
        i = jax.nn.sigmoid(gates[:, 0 * Hd:1 * Hd])
        f = jax.nn.sigmoid(gates[:, 1 * Hd:2 * Hd])
        g = jnp.tanh(gates[:, 2 * Hd:3 * Hd])
        o = jax.nn.sigmoid(gates[:, 3 * Hd:4 * Hd])
        c_new = f * c0_ref[...] + i * g
        h_new = o * jnp.tanh(c_new)
        c_out_ref[...] = c_new
        h_out_ref[...] = h_new
        pv_ref[...] = (jnp.dot(h_new, acw_ref[...],
                               preferred_element_type=jnp.float32)
                       + acb_ref[...])
    return _fused


def kernel(img_pad_value, conv_w_t, conv_b_col, vobs_w, vobs_b, tobs_w, tobs_b,
           lstm_w_ih_v, lstm_w_ih_t, lstm_w_hh, lstm_b, ac_w, ac_b,
           image, glove, h0, c0):
    B, H, W, C = image.shape
    Ho, Wo = H // 2, W // 2
    P = Ho * Wo
    Cout, K = conv_w_t.shape
    Hd = h0.shape[1]
    A1 = ac_w.shape[1]
    bf = jnp.bfloat16

    xp = jnp.pad(image.astype(bf), ((0, 0), (1, 0), (1, 0), (0, 0)))
    slabs = [xp[:, di:di + H:2, dj:dj + W:2, :]
             for di in range(3) for dj in range(3)]
    pat = jnp.stack(slabs, 0)
    pat = pat.transpose(0, 4, 1, 2, 3).reshape(K, B * P)

    ii = jnp.arange(Ho)
    jj = jnp.arange(Wo)
    oob_rows = [((2 * ii + di - 1 < 0) | (2 * ii + di - 1 >= H))
                for di in range(3)]
    oob_cols = [((2 * jj + dj - 1 < 0) | (2 * jj + dj - 1 >= W))
                for dj in range(3)]
    oob = jnp.stack([oob_rows[di][:, None] | oob_cols[dj][None, :]
                     for di in range(3) for dj in range(3)])
    oob27 = jnp.repeat(oob.reshape(9, P).astype(jnp.float32), C, axis=0)
    padv = jnp.tile(img_pad_value.reshape(-1), 9)
    bias = conv_b_col + (conv_w_t * padv[None, :]) @ oob27

    vw3 = vobs_w.reshape(Cout, P, Hd)

    h_new, c_new, pv = pl.pallas_call(
        _make_fused(B, Cout, P, Hd),
        out_shape=(jax.ShapeDtypeStruct((B, Hd), jnp.float32),
                   jax.ShapeDtypeStruct((B, Hd), jnp.float32),
                   jax.ShapeDtypeStruct((B, A1), jnp.float32)),
        in_specs=[_VMEM] * 16,
        out_specs=(_VMEM, _VMEM, _VMEM),
    )(pat, conv_w_t.astype(bf), bias, vw3, vobs_b,
      glove, tobs_w, tobs_b, h0, c0,
      lstm_w_ih_v, lstm_w_ih_t, lstm_w_hh,
      lstm_b, ac_w, ac_b)

    A = A1 - 1
    return {'policy': pv[:, :A], 'value': pv[:, A:], 'hidden': (h_new, c_new)}


# speedup vs baseline: 14.2712x; 4.1348x over previous
import jax
import jax.numpy as jnp
from jax.experimental import pallas as pl
from jax.experimental.pallas import tpu as pltpu

_VMEM = pl.BlockSpec(memory_space=pltpu.MemorySpace.VMEM)


def _make_fused(B, Cout, Ho, Wo, Hd, WC, G):
    P = Ho * Wo
    L = 3 * G

    def _fused(x_ref, m_ref, bias_ref, vw_ref, vb_ref,
               glove_ref, tw_ref, tb_ref, h0_ref, c0_ref,
               wihv_ref, wiht_ref, whh_ref, bl_ref, acw_ref, acb_ref,
               h_out_ref, c_out_ref, pv_ref,
               pad_ref):
        pad_ref[:, 0:1, 0:L] = jnp.zeros((B, 1, L), jnp.bfloat16)
        pad_ref[:, 1:Ho + 1, 0:3] = jnp.zeros((B, Ho, 3), jnp.bfloat16)
        pad_ref[:, 0:Ho, L:L + 3] = jnp.zeros((B, Ho, 3), jnp.bfloat16)
        pad_ref[:, 0:Ho, L + 3:L + 3 + WC] = x_ref[:, :, 0:WC]
        pad_ref[:, 1:Ho + 1, 3:3 + WC] = x_ref[:, :, WC:2 * WC]

        a0 = pad_ref[:, 0:Ho, 0:L].reshape(B * Ho, L)
        a1 = pad_ref[:, 0:Ho, L:2 * L].reshape(B * Ho, L)
        a2 = pad_ref[:, 1:Ho + 1, 0:L].reshape(B * Ho, L)
        cp = (jnp.dot(a0, m_ref[0], preferred_element_type=jnp.float32)
              + jnp.dot(a1, m_ref[1], preferred_element_type=jnp.float32)
              + jnp.dot(a2, m_ref[2], preferred_element_type=jnp.float32))
        conv = jnp.maximum(cp.reshape(B, Ho, Cout * Wo) + bias_ref[...], 0.0)

        acc = jnp.zeros((B, Hd), jnp.float32)
        for c in range(Cout):
            cc = conv[:, :, c * Wo:(c + 1) * Wo].reshape(B, P)
            acc = acc + jnp.dot(cc, vw_ref[c],
                                preferred_element_type=jnp.float32)
        ve = jnp.maximum(acc + vb_ref[...], 0.0)
        te = jnp.maximum(jnp.dot(glove_ref[...], tw_ref[...],
                                 preferred_element_type=jnp.float32)
                         + tb_ref[...], 0.0)
        gates = (jnp.dot(ve, wihv_ref[...],
                         preferred_element_type=jnp.float32)
                 + jnp.dot(te, wiht_ref[...],
                           preferred_element_type=jnp.float32)
                 + jnp.dot(h0_ref[...], whh_ref[...],
                           preferred_element_type=jnp.float32)
                 + bl_ref[...])
        i = jax.nn.sigmoid(gates[:, 0 * Hd:1 * Hd])
        f = jax.nn.sigmoid(gates[:, 1 * Hd:2 * Hd])
        g = jnp.tanh(gates[:, 2 * Hd:3 * Hd])
        o = jax.nn.sigmoid(gates[:, 3 * Hd:4 * Hd])
        c_new = f * c0_ref[...] + i * g
        h_new = o * jnp.tanh(c_new)
        c_out_ref[...] = c_new
        h_out_ref[...] = h_new
        pv_ref[...] = (jnp.dot(h_new, acw_ref[...],
                               preferred_element_type=jnp.float32)
                       + acb_ref[...])
    return _fused


def kernel(img_pad_value, conv_w_t, conv_b_col, vobs_w, vobs_b, tobs_w, tobs_b,
           lstm_w_ih_v, lstm_w_ih_t, lstm_w_hh, lstm_b, ac_w, ac_b,
           image, glove, h0, c0):
    B, H, W, C = image.shape
    Ho, Wo = H // 2, W // 2
    P = Ho * Wo
    Cout, K = conv_w_t.shape
    Hd = h0.shape[1]
    A1 = ac_w.shape[1]
    bf = jnp.bfloat16
    WC = W * C
    G = W + 1
    L = 3 * G

    ximg = image.reshape(B, Ho, 2 * WC).astype(bf)

    jdx = jnp.arange(Wo)
    M = jnp.zeros((3, L, Cout * Wo), jnp.float32)
    for di in range(3):
        acc = jnp.zeros((G, 3, Cout, Wo), jnp.float32)
        for dj in range(3):
            ind = jax.nn.one_hot(2 * jdx + dj, G, axis=0)
            wsl = conv_w_t[:, (di * 3 + dj) * 3:(di * 3 + dj) * 3 + 3]
            acc = acc + jnp.einsum('gj,cd->gdcj', ind, wsl)
        M = M.at[di].set(acc.reshape(L, Cout * Wo))
    M = M.astype(bf)

    ii = jnp.arange(Ho)
    oob_rows = [((2 * ii + di - 1 < 0) | (2 * ii + di - 1 >= H))
                for di in range(3)]
    oob_cols = [((2 * jdx + dj - 1 < 0) | (2 * jdx + dj - 1 >= W))
                for dj in range(3)]
    oob = jnp.stack([oob_rows[di][:, None] | oob_cols[dj][None, :]
                     for di in range(3) for dj in range(3)])
    oob27 = jnp.repeat(oob.reshape(9, P).astype(jnp.float32), C, axis=0)
    padv = jnp.tile(img_pad_value.reshape(-1), 9)
    bias = conv_b_col + (conv_w_t * padv[None, :]) @ oob27
    biasN = bias.reshape(Cout, Ho, Wo).transpose(1, 0, 2).reshape(1, Ho, Cout * Wo)

    vw3 = vobs_w.reshape(Cout, P, Hd)

    h_new, c_new, pv = pl.pallas_call(
        _make_fused(B, Cout, Ho, Wo, Hd, WC, G),
        out_shape=(jax.ShapeDtypeStruct((B, Hd), jnp.float32),
                   jax.ShapeDtypeStruct((B, Hd), jnp.float32),
                   jax.ShapeDtypeStruct((B, A1), jnp.float32)),
        in_specs=[_VMEM] * 16,
        out_specs=(_VMEM, _VMEM, _VMEM),
        scratch_shapes=[pltpu.VMEM((B, Ho + 1, 2 * L), bf)],
    )(ximg, M, biasN, vw3, vobs_b,
      glove, tobs_w, tobs_b, h0, c0,
      lstm_w_ih_v, lstm_w_ih_t, lstm_w_hh,
      lstm_b, ac_w, ac_b)

    A = A1 - 1
    return {'policy': pv[:, :A], 'value': pv[:, A:], 'hidden': (h_new, c_new)}
